# Initial kernel scaffold; baseline (speedup 1.0000x reference)
import jax
import jax.numpy as jnp
from jax.experimental import pallas as pl


def kernel(features, labels):
    raise NotImplementedError("write your pallas kernel here")


# speedup vs baseline: 1.5205x; 1.5205x over previous
import jax
import jax.numpy as jnp
from jax import lax
from jax.experimental import pallas as pl
from jax.experimental.pallas import tpu as pltpu
from jax.experimental.pallas import tpu_sc as plsc

_NCLASS = 1000
_D = 512
_B = 16384
_NC = 2
_NS = 16
_NT = _NC * _NS
_L = 16
_CPT = 32
_G = 16
_NVEC = _D // _L


def _sc_body(feat_hbm, lab_hbm, out_hbm,
             labs_v, rowbuf, labbuf, rows_v, tab_v, cnt_v, out_v):
    c = lax.axis_index("c")
    s = lax.axis_index("s")
    t = s * _NC + c
    zvec = jnp.zeros((_L,), jnp.float32)
    ovec = jnp.ones((_L,), jnp.float32)
    iota = lax.iota(jnp.int32, _L)
    five = jnp.full((_L,), 5, jnp.int32)
    onei = jnp.full((_L,), 1, jnp.int32)
    tvec = jnp.broadcast_to(t, (_L,))
    cvec0 = jnp.broadcast_to(t * _CPT, (_L,))

    pltpu.sync_copy(lab_hbm, labs_v)

    def zrow(i, _):
        for cc in range(_NVEC):
            tab_v[i, pl.ds(cc * _L, _L)] = zvec
        cnt_v[i] = 0.0
        return 0

    lax.fori_loop(0, _CPT, zrow, 0)
    out_v[0] = zvec
    out_v[1] = zvec

    def scan_body(i, off):
        lv = labs_v[pl.ds(i * _L, _L)]
        m = lax.shift_right_logical(lv, five) == tvec
        rid = iota + jnp.broadcast_to(i * _L, (_L,))
        cs = plsc.cumsum(jnp.where(m, onei, onei - onei))
        pos = (cs - onei) + jnp.broadcast_to(off, (_L,))
        plsc.store_scatter(rowbuf, [pos], rid, mask=m)
        plsc.store_scatter(labbuf, [pos], lv, mask=m)
        return off + cs[_L - 1]

    n = lax.fori_loop(0, _B // _L, scan_body, jnp.int32(0))
    rowbuf[pl.ds(n, _L)] = jnp.zeros((_L,), jnp.int32)
    labbuf[pl.ds(n, _L)] = cvec0

    def bulk_chunk(cid, ssq):
        pltpu.sync_copy(feat_hbm.at[rowbuf.at[pl.ds(cid * _G, _G)]], rows_v)
        lvec = labbuf[pl.ds(cid * _G, _L)] - cvec0
        acc = ssq
        for r in range(_G):
            lc = lvec[r]
            cnt_v[lc] = cnt_v[lc] + 1.0
            for cc in range(_NVEC):
                v = rows_v[r, pl.ds(cc * _L, _L)]
                plsc.addupdate(tab_v.at[lc, pl.ds(cc * _L, _L)], v)
                acc = acc + v * v
        return acc

    nfull = n // _G
    ssq = lax.fori_loop(0, nfull, bulk_chunk, zvec)
    out_v[0] = ssq

    @pl.when(nfull * _G < n)
    def _tail():
        base = nfull * _G
        pltpu.sync_copy(feat_hbm.at[rowbuf.at[pl.ds(base, _G)]], rows_v)
        lvec = labbuf[pl.ds(base, _L)] - cvec0
        for r in range(_G):
            q = base + r
            f = jnp.where(q < n, 1.0, 0.0)
            fv = jnp.broadcast_to(f, (_L,))
            lc = lvec[r]
            cnt_v[lc] = cnt_v[lc] + f
            for cc in range(_NVEC):
                v = rows_v[r, pl.ds(cc * _L, _L)] * fv
                plsc.addupdate(tab_v.at[lc, pl.ds(cc * _L, _L)], v)
                plsc.addupdate(out_v.at[0], v * v)

    def crow(lc, ctr):
        cvec = jnp.broadcast_to(cnt_v[lc], (_L,))
        inv = ovec / jnp.maximum(cvec, ovec)
        rowacc = zvec
        for cc in range(_NVEC):
            v = tab_v[lc, pl.ds(cc * _L, _L)]
            rowacc = rowacc + v * v
        return ctr + rowacc * inv

    ctr = lax.fori_loop(0, _CPT, crow, zvec)
    out_v[1] = ctr
    pltpu.sync_copy(out_v, out_hbm.at[t])


@jax.jit
def _center_loss_sc(features, labels):
    parts = pl.kernel(
        _sc_body,
        out_type=jax.ShapeDtypeStruct((_NT, 2, _L), jnp.float32),
        mesh=plsc.VectorSubcoreMesh(core_axis_name="c", subcore_axis_name="s"),
        compiler_params=pltpu.CompilerParams(needs_layout_passes=False),
        scratch_types=[
            pltpu.VMEM((_B,), jnp.int32),
            pltpu.VMEM((_B + _L,), jnp.int32),
            pltpu.VMEM((_B + _L,), jnp.int32),
            pltpu.VMEM((_G, _D), jnp.float32),
            pltpu.VMEM((_CPT, _D), jnp.float32),
            pltpu.SMEM((_CPT,), jnp.float32),
            pltpu.VMEM((2, _L), jnp.float32),
        ],
    )(features, labels)
    ssq = jnp.sum(parts[:, 0, :])
    ctr = jnp.sum(parts[:, 1, :])
    return (ssq - ctr) / (2.0 * features.shape[0])


def kernel(features, labels):
    return _center_loss_sc(features, labels)
